# Initial kernel scaffold; baseline (speedup 1.0000x reference)
import jax
import jax.numpy as jnp
from jax.experimental import pallas as pl


def kernel(node_feat, adj):
    raise NotImplementedError("write your pallas kernel here")


# speedup vs baseline: 1.3574x; 1.3574x over previous
import functools

import jax
import jax.numpy as jnp
from jax.experimental import pallas as pl
from jax.experimental.pallas import tpu as pltpu


def _wls_body(adj_ref, feat_ref, o_ref, *, inv_m):
    adj_sum = jnp.sum(adj_ref[...], axis=1)
    feat = feat_ref[...]
    a16 = adj_sum.astype(jnp.bfloat16)
    f16 = (feat * inv_m).astype(jnp.bfloat16)
    agg = jax.lax.dot_general(
        a16, f16,
        dimension_numbers=(((2,), (1,)), ((0,), (0,))),
        preferred_element_type=jnp.float32,
    )
    o_ref[...] = feat + agg


def kernel(node_feat, adj):
    B, N, D = node_feat.shape
    _, M, _, _ = adj.shape
    inv_m = 1.0 / float(M)

    block_b = 8
    grid = (B // block_b,)
    return pl.pallas_call(
        functools.partial(_wls_body, inv_m=inv_m),
        out_shape=jax.ShapeDtypeStruct((B, N, D), node_feat.dtype),
        grid=grid,
        in_specs=[
            pl.BlockSpec((block_b, M, N, N), lambda b: (b, 0, 0, 0)),
            pl.BlockSpec((block_b, N, D), lambda b: (b, 0, 0)),
        ],
        out_specs=pl.BlockSpec((block_b, N, D), lambda b: (b, 0, 0)),
        compiler_params=pltpu.CompilerParams(
            dimension_semantics=("parallel",),
            vmem_limit_bytes=64 * 1024 * 1024,
        ),
    )(adj, node_feat)
